# Initial kernel scaffold; baseline (speedup 1.0000x reference)
import jax
import jax.numpy as jnp
from jax.experimental import pallas as pl


def kernel(data, e_table, r_table):
    raise NotImplementedError("write your pallas kernel here")


# speedup vs baseline: 1.1940x; 1.1940x over previous
import functools

import jax
import jax.numpy as jnp
from jax import lax
from jax.experimental import pallas as pl
from jax.experimental.pallas import tpu as pltpu
from jax.experimental.pallas import tpu_sc as plsc

B = 1024
N = 64
D = 128
TOTAL = B * N
NC, NS, L = 2, 16, 16
NW = NC * NS
PER_W = TOTAL // NW
C = 128
NCH = PER_W // C


def _sc_body(h_hbm, r_hbm, t_hbm, e_hbm, rel_hbm, out_hbm,
             hi_v, ri_v, ti_v, h_rows, r_rows, t_rows, part_v, out_v, sem):
    wid = lax.axis_index("s") * NC + lax.axis_index("c")
    base = wid * PER_W
    lane_iota = lax.iota(jnp.int32, L)

    for c in range(NCH):
        off = base + c * C
        pltpu.sync_copy(h_hbm.at[pl.ds(off, C)], hi_v)
        pltpu.sync_copy(r_hbm.at[pl.ds(off, C)], ri_v)
        pltpu.sync_copy(t_hbm.at[pl.ds(off, C)], ti_v)
        cp_h = pltpu.async_copy(e_hbm.at[hi_v], h_rows, sem)
        cp_r = pltpu.async_copy(rel_hbm.at[ri_v], r_rows, sem)
        cp_t = pltpu.async_copy(e_hbm.at[ti_v], t_rows, sem)
        cp_h.wait()
        cp_r.wait()
        cp_t.wait()

        def group_body(g, _, c=c):
            def row_body(rr, _):
                i = g * L + rr
                acc = (h_rows[i, pl.ds(0, L)]
                       * r_rows[i, pl.ds(0, L)]
                       * t_rows[i, pl.ds(0, L)])
                for j in range(1, D // L):
                    acc = acc + (h_rows[i, pl.ds(j * L, L)]
                                 * r_rows[i, pl.ds(j * L, L)]
                                 * t_rows[i, pl.ds(j * L, L)])
                plsc.store_scatter(part_v, [lane_iota * L + rr], acc)
                return 0

            lax.fori_loop(0, L, row_body, 0)

            tot = part_v[pl.ds(0, L)]
            for k in range(1, L):
                tot = tot + part_v[pl.ds(k * L, L)]
            out_v[pl.ds(c * C + g * L, L)] = tot
            return 0

        lax.fori_loop(0, C // L, group_body, 0)

    pltpu.sync_copy(out_v, out_hbm.at[pl.ds(base, PER_W)])


@functools.partial(
    pl.kernel,
    out_type=jax.ShapeDtypeStruct((TOTAL,), jnp.float32),
    mesh=plsc.VectorSubcoreMesh(core_axis_name="c", subcore_axis_name="s"),
    compiler_params=pltpu.CompilerParams(needs_layout_passes=False),
    scratch_types=[
        pltpu.VMEM((C,), jnp.int32),
        pltpu.VMEM((C,), jnp.int32),
        pltpu.VMEM((C,), jnp.int32),
        pltpu.VMEM((C, D), jnp.float32),
        pltpu.VMEM((C, D), jnp.float32),
        pltpu.VMEM((C, D), jnp.float32),
        pltpu.VMEM((L * L,), jnp.float32),
        pltpu.VMEM((PER_W,), jnp.float32),
        pltpu.SemaphoreType.DMA,
    ],
)
def _distmult_sc(h_hbm, r_hbm, t_hbm, e_hbm, rel_hbm, out_hbm,
                 hi_v, ri_v, ti_v, h_rows, r_rows, t_rows, part_v, out_v,
                 sem):
    _sc_body(h_hbm, r_hbm, t_hbm, e_hbm, rel_hbm, out_hbm,
             hi_v, ri_v, ti_v, h_rows, r_rows, t_rows, part_v, out_v, sem)


def kernel(data, e_table, r_table):
    flat = data.reshape(TOTAL, 3)
    h_idx = flat[:, 0].astype(jnp.int32)
    r_idx = flat[:, 1].astype(jnp.int32)
    t_idx = flat[:, 2].astype(jnp.int32)
    out = _distmult_sc(h_idx, r_idx, t_idx, e_table, r_table)
    return out.reshape(B, N)


# speedup vs baseline: 2.1063x; 1.7641x over previous
import functools

import jax
import jax.numpy as jnp
from jax import lax
from jax.experimental import pallas as pl
from jax.experimental.pallas import tpu as pltpu
from jax.experimental.pallas import tpu_sc as plsc

B = 1024
N = 64
D = 128
TOTAL = B * N
NC, NS, L = 2, 16, 16
NW = NC * NS
PER_W = TOTAL // NW
C = 128
NCH = PER_W // C


def _compute_chunk(h_rows, r_rows, t_rows, part_v, out_v, c, lane_iota):
    def group_body(g, _):
        def row_body(rr, _):
            i = g * L + rr
            acc = (h_rows[i, pl.ds(0, L)]
                   * r_rows[i, pl.ds(0, L)]
                   * t_rows[i, pl.ds(0, L)])
            for j in range(1, D // L):
                acc = acc + (h_rows[i, pl.ds(j * L, L)]
                             * r_rows[i, pl.ds(j * L, L)]
                             * t_rows[i, pl.ds(j * L, L)])
            plsc.store_scatter(part_v, [lane_iota * L + rr], acc)
            return 0

        lax.fori_loop(0, L, row_body, 0)

        tot = part_v[pl.ds(0, L)]
        for k in range(1, L):
            tot = tot + part_v[pl.ds(k * L, L)]
        out_v[pl.ds(c * C + g * L, L)] = tot
        return 0

    lax.fori_loop(0, C // L, group_body, 0)


def _sc_body(h_hbm, r_hbm, t_hbm, e_hbm, rel_hbm, out_hbm,
             hi_all, ri_all, ti_all, bufs, part_v, out_v, sem_idx, sems):
    wid = lax.axis_index("s") * NC + lax.axis_index("c")
    base = wid * PER_W
    lane_iota = lax.iota(jnp.int32, L)

    cps = [pltpu.async_copy(h_hbm.at[pl.ds(base, PER_W)], hi_all, sem_idx),
           pltpu.async_copy(r_hbm.at[pl.ds(base, PER_W)], ri_all, sem_idx),
           pltpu.async_copy(t_hbm.at[pl.ds(base, PER_W)], ti_all, sem_idx)]
    for cp in cps:
        cp.wait()

    def fire(c, b):
        sl = pl.ds(c * C, C)
        h_rows, r_rows, t_rows = bufs[b]
        return [
            pltpu.async_copy(e_hbm.at[hi_all.at[sl]], h_rows, sems[b]),
            pltpu.async_copy(rel_hbm.at[ri_all.at[sl]], r_rows, sems[b]),
            pltpu.async_copy(e_hbm.at[ti_all.at[sl]], t_rows, sems[b]),
        ]

    pending = {0: fire(0, 0)}
    for c in range(NCH):
        b = c % 2
        if c + 1 < NCH:
            pending[c + 1] = fire(c + 1, 1 - b)
        for cp in pending.pop(c):
            cp.wait()
        h_rows, r_rows, t_rows = bufs[b]
        _compute_chunk(h_rows, r_rows, t_rows, part_v, out_v, c, lane_iota)

    pltpu.sync_copy(out_v, out_hbm.at[pl.ds(base, PER_W)])


@functools.partial(
    pl.kernel,
    out_type=jax.ShapeDtypeStruct((TOTAL,), jnp.float32),
    mesh=plsc.VectorSubcoreMesh(core_axis_name="c", subcore_axis_name="s"),
    compiler_params=pltpu.CompilerParams(needs_layout_passes=False),
    scratch_types=[
        pltpu.VMEM((PER_W,), jnp.int32),
        pltpu.VMEM((PER_W,), jnp.int32),
        pltpu.VMEM((PER_W,), jnp.int32),
        pltpu.VMEM((C, D), jnp.float32),
        pltpu.VMEM((C, D), jnp.float32),
        pltpu.VMEM((C, D), jnp.float32),
        pltpu.VMEM((C, D), jnp.float32),
        pltpu.VMEM((C, D), jnp.float32),
        pltpu.VMEM((C, D), jnp.float32),
        pltpu.VMEM((L * L,), jnp.float32),
        pltpu.VMEM((PER_W,), jnp.float32),
        pltpu.SemaphoreType.DMA,
        pltpu.SemaphoreType.DMA,
        pltpu.SemaphoreType.DMA,
    ],
)
def _distmult_sc(h_hbm, r_hbm, t_hbm, e_hbm, rel_hbm, out_hbm,
                 hi_all, ri_all, ti_all,
                 h0, r0, t0, h1, r1, t1, part_v, out_v,
                 sem_idx, sem_a, sem_b):
    _sc_body(h_hbm, r_hbm, t_hbm, e_hbm, rel_hbm, out_hbm,
             hi_all, ri_all, ti_all,
             [(h0, r0, t0), (h1, r1, t1)], part_v, out_v,
             sem_idx, [sem_a, sem_b])


def kernel(data, e_table, r_table):
    flat = data.reshape(TOTAL, 3)
    h_idx = flat[:, 0].astype(jnp.int32)
    r_idx = flat[:, 1].astype(jnp.int32)
    t_idx = flat[:, 2].astype(jnp.int32)
    out = _distmult_sc(h_idx, r_idx, t_idx, e_table, r_table)
    return out.reshape(B, N)
